# baseline (speedup 1.0000x reference)
import functools

import jax
import jax.numpy as jnp
from jax import lax
from jax.experimental import pallas as pl
from jax.experimental.pallas import tpu as pltpu
from jax.experimental.pallas import tpu_sc as plsc

_REG = 0.001
_B = 16384
_D = 64
_NC = 2
_NS = 16
_L = 16
_NW = _NC * _NS
_BPW = _B // _NW
_CHUNK = 64
_NCHUNK = _BPW // _CHUNK


def _sc_body(user_hbm, ii_hbm, ij_hbm, eu2_hbm, ei2_hbm,
             pi_hbm, pj_hbm, reg_hbm,
             uidx_v, iidx_v, jidx_v, upr_v, ipr_v, jpr_v,
             u2a, vi2a, vj2a, u2b, vi2b, vj2b, pi_v, pj_v, reg_v,
             sem_a, sem_b):
    c = lax.axis_index("c")
    s = lax.axis_index("s")
    wid = s * _NC + c
    base = wid * _BPW

    pltpu.sync_copy(user_hbm.at[pl.ds(base, _BPW)], uidx_v)
    pltpu.sync_copy(ii_hbm.at[pl.ds(base, _BPW)], iidx_v)
    pltpu.sync_copy(ij_hbm.at[pl.ds(base, _BPW)], jidx_v)

    def pair_row(v):
        return lax.bitwise_or(
            lax.shift_left(lax.shift_right_logical(v, 4), 3),
            lax.bitwise_and(v, 7))

    def halve(i, _):
        sl = pl.ds(i * _L, _L)
        upr_v[sl] = pair_row(uidx_v[sl])
        ipr_v[sl] = pair_row(iidx_v[sl])
        jpr_v[sl] = pair_row(jidx_v[sl])
        return 0
    lax.fori_loop(0, _BPW // _L, halve, 0)

    lane = lax.iota(jnp.int32, _L)
    reg_acc0 = jnp.zeros((_L,), jnp.float32)

    def make_group(chunk_off, u2_v, vi2_v, vj2_v):
        def group(g, reg_acc):
            goff = chunk_off + g * _L
            def half_off(v):
                return lax.bitwise_and(lax.shift_right_logical(v, 3), 1) * _D
            pu = half_off(uidx_v[pl.ds(goff, _L)])
            pi_ = half_off(iidx_v[pl.ds(goff, _L)])
            pj_ = half_off(jidx_v[pl.ds(goff, _L)])
            acc_i = jnp.zeros((_L,), jnp.float32)
            acc_j = jnp.zeros((_L,), jnp.float32)
            for l in range(_L):
                r = g * _L + l
                hu = pu[l]
                hi = pi_[l]
                hj = pj_[l]
                ss_i = jnp.zeros((_L,), jnp.float32)
                ss_j = jnp.zeros((_L,), jnp.float32)
                for cc in range(_D // _L):
                    uu = u2_v[r, pl.ds(hu + cc * _L, _L)]
                    vv = vi2_v[r, pl.ds(hi + cc * _L, _L)]
                    ww = vj2_v[r, pl.ds(hj + cc * _L, _L)]
                    ss_i = ss_i + uu * vv
                    ss_j = ss_j + uu * ww
                    reg_acc = reg_acc + uu * uu + vv * vv + ww * ww
                acc_i = jnp.where(lane == l, jnp.sum(ss_i), acc_i)
                acc_j = jnp.where(lane == l, jnp.sum(ss_j), acc_j)
            row = pl.ds(chunk_off + g * _L, _L)
            pi_v[row] = acc_i
            pj_v[row] = acc_j
            return reg_acc
        return group

    slots = ((u2a, vi2a, vj2a, sem_a), (u2b, vi2b, vj2b, sem_b))

    def fire(k, slot):
        u2_v, vi2_v, vj2_v, sem = slot
        sl = pl.ds(k * _CHUNK, _CHUNK)
        return (pltpu.async_copy(eu2_hbm.at[upr_v.at[sl]], u2_v, sem),
                pltpu.async_copy(ei2_hbm.at[ipr_v.at[sl]], vi2_v, sem),
                pltpu.async_copy(ei2_hbm.at[jpr_v.at[sl]], vj2_v, sem))

    reg_acc = reg_acc0
    inflight = fire(0, slots[0])
    for k in range(_NCHUNK):
        slot = slots[k % 2]
        for cp in inflight:
            cp.wait()
        if k + 1 < _NCHUNK:
            inflight = fire(k + 1, slots[(k + 1) % 2])
        reg_acc = lax.fori_loop(
            0, _CHUNK // _L, make_group(k * _CHUNK, *slot[:3]), reg_acc)

    reg_v[...] = reg_acc
    pltpu.sync_copy(pi_v, pi_hbm.at[pl.ds(base, _BPW)])
    pltpu.sync_copy(pj_v, pj_hbm.at[pl.ds(base, _BPW)])
    pltpu.sync_copy(reg_v, reg_hbm.at[pl.ds(wid * _L, _L)])


_sc_call = functools.partial(
    pl.kernel,
    out_type=(
        jax.ShapeDtypeStruct((_B,), jnp.float32),
        jax.ShapeDtypeStruct((_B,), jnp.float32),
        jax.ShapeDtypeStruct((_NW * _L,), jnp.float32),
    ),
    mesh=plsc.VectorSubcoreMesh(
        core_axis_name="c", subcore_axis_name="s",
        num_cores=_NC, num_subcores=_NS),
    compiler_params=pltpu.CompilerParams(
        needs_layout_passes=False, use_tc_tiling_on_sc=True),
    scratch_types=[
        pltpu.VMEM((_BPW,), jnp.int32),
        pltpu.VMEM((_BPW,), jnp.int32),
        pltpu.VMEM((_BPW,), jnp.int32),
        pltpu.VMEM((_BPW,), jnp.int32),
        pltpu.VMEM((_BPW,), jnp.int32),
        pltpu.VMEM((_BPW,), jnp.int32),
        pltpu.VMEM((_CHUNK, 2 * _D), jnp.float32),
        pltpu.VMEM((_CHUNK, 2 * _D), jnp.float32),
        pltpu.VMEM((_CHUNK, 2 * _D), jnp.float32),
        pltpu.VMEM((_CHUNK, 2 * _D), jnp.float32),
        pltpu.VMEM((_CHUNK, 2 * _D), jnp.float32),
        pltpu.VMEM((_CHUNK, 2 * _D), jnp.float32),
        pltpu.VMEM((_BPW,), jnp.float32),
        pltpu.VMEM((_BPW,), jnp.float32),
        pltpu.VMEM((_L,), jnp.float32),
        pltpu.SemaphoreType.DMA,
        pltpu.SemaphoreType.DMA,
    ],
)(_sc_body)


def _pair_body(t_ref, o_ref):
    x = jnp.swapaxes(t_ref[...], 0, 1)
    q = x.shape[0] // 16
    x4 = x.reshape(q, 2, 8, _D)
    out = jnp.concatenate([x4[:, 0], x4[:, 1]], axis=-1)
    o_ref[...] = out.reshape(q * 8, 2 * _D)


def _pair_rows(table):
    n = table.shape[0]
    m = 8 * ((n + 15) // 16)
    blk = 8192
    grid = (m + blk - 1) // blk
    return pl.pallas_call(
        _pair_body,
        grid=(grid,),
        in_specs=[pl.BlockSpec((_D, 2 * blk), lambda i: (0, i))],
        out_specs=pl.BlockSpec((blk, 2 * _D), lambda i: (i, 0)),
        out_shape=jax.ShapeDtypeStruct((m, 2 * _D), table.dtype),
    )(table.T)


def _loss_body(pi_ref, pj_ref, reg_ref, out_ref):
    x = pi_ref[...] - pj_ref[...]
    ls = jnp.minimum(x, 0.0) - jnp.log(1.0 + jnp.exp(-jnp.abs(x)))
    out_ref[0, 0] = _REG * jnp.sum(reg_ref[...]) - jnp.sum(ls)


_loss_call = pl.pallas_call(
    _loss_body,
    out_shape=jax.ShapeDtypeStruct((1, 1), jnp.float32),
    out_specs=pl.BlockSpec(memory_space=pltpu.SMEM),
)


def kernel(user, item_i, item_j, embed_user, embed_item):
    eu2 = _pair_rows(embed_user)
    ei2 = _pair_rows(embed_item)
    pi, pj, regp = _sc_call(user, item_i, item_j, eu2, ei2)
    loss = _loss_call(pi.reshape(_B // 128, 128),
                      pj.reshape(_B // 128, 128),
                      regp.reshape(_NW * _L // 128, 128))[0, 0]
    return (pi, pj, loss)


# speedup vs baseline: 1.0203x; 1.0203x over previous
import functools

import jax
import jax.numpy as jnp
from jax import lax
from jax.experimental import pallas as pl
from jax.experimental.pallas import tpu as pltpu
from jax.experimental.pallas import tpu_sc as plsc

_REG = 0.001
_B = 16384
_D = 64
_NC = 2
_NS = 16
_L = 16
_NW = _NC * _NS
_BPW = _B // _NW
_CHUNK = 128
_NCHUNK = _BPW // _CHUNK

_sc_mesh = plsc.VectorSubcoreMesh(
    core_axis_name="c", subcore_axis_name="s",
    num_cores=_NC, num_subcores=_NS)
_sc_params = pltpu.CompilerParams(
    needs_layout_passes=False, use_tc_tiling_on_sc=True)


def _pair_row(v):
    return lax.bitwise_or(
        lax.shift_left(lax.shift_right_logical(v, 4), 3),
        lax.bitwise_and(v, 7))


def _half_off(v):
    return lax.bitwise_and(lax.shift_right_logical(v, 3), 1) * _D


def _sc_item_body(ii_hbm, ij_hbm, ei2_hbm,
                  svi_hbm, svj_hbm, reg_hbm,
                  iidx_v, jidx_v, ipr_v, jpr_v,
                  via, vja, vib, vjb, cvi, cvj, reg_v,
                  sem_a, sem_b, sem_w):
    c = lax.axis_index("c")
    s = lax.axis_index("s")
    wid = s * _NC + c
    base = wid * _BPW
    sbase = wid * (_BPW // 2)

    pltpu.sync_copy(ii_hbm.at[pl.ds(base, _BPW)], iidx_v)
    pltpu.sync_copy(ij_hbm.at[pl.ds(base, _BPW)], jidx_v)

    def prep(i, _):
        sl = pl.ds(i * _L, _L)
        ipr_v[sl] = _pair_row(iidx_v[sl])
        jpr_v[sl] = _pair_row(jidx_v[sl])
        return 0
    lax.fori_loop(0, _BPW // _L, prep, 0)

    slots = ((via, vja, sem_a), (vib, vjb, sem_b))

    def fire(k, slot):
        vi_v, vj_v, sem = slot
        sl = pl.ds(k * _CHUNK, _CHUNK)
        return (pltpu.async_copy(ei2_hbm.at[ipr_v.at[sl]], vi_v, sem),
                pltpu.async_copy(ei2_hbm.at[jpr_v.at[sl]], vj_v, sem))

    reg_acc = jnp.zeros((_L,), jnp.float32)
    inflight = fire(0, slots[0])
    writes = []
    for k in range(_NCHUNK):
        vi_v, vj_v, _ = slots[k % 2]
        for cp in inflight:
            cp.wait()
        if k + 1 < _NCHUNK:
            inflight = fire(k + 1, slots[(k + 1) % 2])

        def compact(g, reg_acc, vi_v=vi_v, vj_v=vj_v, k=k):
            goff = k * _CHUNK + g * _L
            hi = _half_off(iidx_v[pl.ds(goff, _L)])
            hj = _half_off(jidx_v[pl.ds(goff, _L)])
            for l in range(_L):
                r = g * _L + l
                hoi = hi[l]
                hoj = hj[l]
                dst = (r & 1) * _D
                for cc in range(_D // _L):
                    col = pl.ds(dst + cc * _L, _L)
                    vv = vi_v[r, pl.ds(hoi + cc * _L, _L)]
                    ww = vj_v[r, pl.ds(hoj + cc * _L, _L)]
                    cvi[r // 2, col] = vv
                    cvj[r // 2, col] = ww
                    reg_acc = reg_acc + vv * vv + ww * ww
            return reg_acc

        reg_acc = lax.fori_loop(0, _CHUNK // _L, compact, reg_acc)
        dst_sl = pl.ds(sbase + k * (_CHUNK // 2), _CHUNK // 2)
        writes.append(pltpu.async_copy(cvi, svi_hbm.at[dst_sl], sem_w))
        writes.append(pltpu.async_copy(cvj, svj_hbm.at[dst_sl], sem_w))
        for cp in writes:
            cp.wait()
        writes = []

    reg_v[...] = reg_acc
    pltpu.sync_copy(reg_v, reg_hbm.at[pl.ds(wid * _L, _L)])


_sc_item = functools.partial(
    pl.kernel,
    out_type=(
        jax.ShapeDtypeStruct((_B // 2, 2 * _D), jnp.float32),
        jax.ShapeDtypeStruct((_B // 2, 2 * _D), jnp.float32),
        jax.ShapeDtypeStruct((_NW * _L,), jnp.float32),
    ),
    mesh=_sc_mesh,
    compiler_params=_sc_params,
    scratch_types=[
        pltpu.VMEM((_BPW,), jnp.int32),
        pltpu.VMEM((_BPW,), jnp.int32),
        pltpu.VMEM((_BPW,), jnp.int32),
        pltpu.VMEM((_BPW,), jnp.int32),
        pltpu.VMEM((_CHUNK, 2 * _D), jnp.float32),
        pltpu.VMEM((_CHUNK, 2 * _D), jnp.float32),
        pltpu.VMEM((_CHUNK, 2 * _D), jnp.float32),
        pltpu.VMEM((_CHUNK, 2 * _D), jnp.float32),
        pltpu.VMEM((_CHUNK // 2, 2 * _D), jnp.float32),
        pltpu.VMEM((_CHUNK // 2, 2 * _D), jnp.float32),
        pltpu.VMEM((_L,), jnp.float32),
        pltpu.SemaphoreType.DMA,
        pltpu.SemaphoreType.DMA,
        pltpu.SemaphoreType.DMA,
    ],
)(_sc_item_body)


def _sc_user_body(user_hbm, eu2_hbm, svi_hbm, svj_hbm,
                  pi_hbm, pj_hbm, reg_hbm,
                  uidx_v, upr_v,
                  ua, ub, via, vja, vib, vjb, pi_v, pj_v, reg_v,
                  sem_a, sem_b):
    c = lax.axis_index("c")
    s = lax.axis_index("s")
    wid = s * _NC + c
    base = wid * _BPW
    sbase = wid * (_BPW // 2)

    pltpu.sync_copy(user_hbm.at[pl.ds(base, _BPW)], uidx_v)

    def prep(i, _):
        sl = pl.ds(i * _L, _L)
        upr_v[sl] = _pair_row(uidx_v[sl])
        return 0
    lax.fori_loop(0, _BPW // _L, prep, 0)

    slots = ((ua, via, vja, sem_a), (ub, vib, vjb, sem_b))

    def fire(k, slot):
        u_v, vi_v, vj_v, sem = slot
        sl = pl.ds(k * _CHUNK, _CHUNK)
        ssl = pl.ds(sbase + k * (_CHUNK // 2), _CHUNK // 2)
        return (pltpu.async_copy(eu2_hbm.at[upr_v.at[sl]], u_v, sem),
                pltpu.async_copy(svi_hbm.at[ssl], vi_v, sem),
                pltpu.async_copy(svj_hbm.at[ssl], vj_v, sem))

    lane = lax.iota(jnp.int32, _L)
    reg_acc = jnp.zeros((_L,), jnp.float32)
    inflight = fire(0, slots[0])
    for k in range(_NCHUNK):
        u_v, vi_v, vj_v, _ = slots[k % 2]
        for cp in inflight:
            cp.wait()
        if k + 1 < _NCHUNK:
            inflight = fire(k + 1, slots[(k + 1) % 2])

        def group(g, reg_acc, u_v=u_v, vi_v=vi_v, vj_v=vj_v, k=k):
            goff = k * _CHUNK + g * _L
            hu = _half_off(uidx_v[pl.ds(goff, _L)])
            acc_i = jnp.zeros((_L,), jnp.float32)
            acc_j = jnp.zeros((_L,), jnp.float32)
            for l in range(_L):
                r = g * _L + l
                hou = hu[l]
                hst = (r & 1) * _D
                ss_i = jnp.zeros((_L,), jnp.float32)
                ss_j = jnp.zeros((_L,), jnp.float32)
                for cc in range(_D // _L):
                    uu = u_v[r, pl.ds(hou + cc * _L, _L)]
                    vv = vi_v[r // 2, pl.ds(hst + cc * _L, _L)]
                    ww = vj_v[r // 2, pl.ds(hst + cc * _L, _L)]
                    ss_i = ss_i + uu * vv
                    ss_j = ss_j + uu * ww
                    reg_acc = reg_acc + uu * uu
                acc_i = jnp.where(lane == l, jnp.sum(ss_i), acc_i)
                acc_j = jnp.where(lane == l, jnp.sum(ss_j), acc_j)
            row = pl.ds(k * _CHUNK + g * _L, _L)
            pi_v[row] = acc_i
            pj_v[row] = acc_j
            return reg_acc

        reg_acc = lax.fori_loop(0, _CHUNK // _L, group, reg_acc)

    reg_v[...] = reg_acc
    pltpu.sync_copy(pi_v, pi_hbm.at[pl.ds(base, _BPW)])
    pltpu.sync_copy(pj_v, pj_hbm.at[pl.ds(base, _BPW)])
    pltpu.sync_copy(reg_v, reg_hbm.at[pl.ds(wid * _L, _L)])


_sc_user = functools.partial(
    pl.kernel,
    out_type=(
        jax.ShapeDtypeStruct((_B,), jnp.float32),
        jax.ShapeDtypeStruct((_B,), jnp.float32),
        jax.ShapeDtypeStruct((_NW * _L,), jnp.float32),
    ),
    mesh=_sc_mesh,
    compiler_params=_sc_params,
    scratch_types=[
        pltpu.VMEM((_BPW,), jnp.int32),
        pltpu.VMEM((_BPW,), jnp.int32),
        pltpu.VMEM((_CHUNK, 2 * _D), jnp.float32),
        pltpu.VMEM((_CHUNK, 2 * _D), jnp.float32),
        pltpu.VMEM((_CHUNK // 2, 2 * _D), jnp.float32),
        pltpu.VMEM((_CHUNK // 2, 2 * _D), jnp.float32),
        pltpu.VMEM((_CHUNK // 2, 2 * _D), jnp.float32),
        pltpu.VMEM((_CHUNK // 2, 2 * _D), jnp.float32),
        pltpu.VMEM((_BPW,), jnp.float32),
        pltpu.VMEM((_BPW,), jnp.float32),
        pltpu.VMEM((_L,), jnp.float32),
        pltpu.SemaphoreType.DMA,
        pltpu.SemaphoreType.DMA,
    ],
)(_sc_user_body)


def _pair_body(t_ref, o_ref):
    x = jnp.swapaxes(t_ref[...], 0, 1)
    q = x.shape[0] // 16
    x4 = x.reshape(q, 2, 8, _D)
    out = jnp.concatenate([x4[:, 0], x4[:, 1]], axis=-1)
    o_ref[...] = out.reshape(q * 8, 2 * _D)


def _pair_rows(table):
    n = table.shape[0]
    m = 8 * ((n + 15) // 16)
    blk = 8192
    grid = (m + blk - 1) // blk
    return pl.pallas_call(
        _pair_body,
        grid=(grid,),
        in_specs=[pl.BlockSpec((_D, 2 * blk), lambda i: (0, i))],
        out_specs=pl.BlockSpec((blk, 2 * _D), lambda i: (i, 0)),
        out_shape=jax.ShapeDtypeStruct((m, 2 * _D), table.dtype),
    )(table.T)


def _loss_body(pi_ref, pj_ref, regi_ref, regu_ref, out_ref):
    x = pi_ref[...] - pj_ref[...]
    ls = jnp.minimum(x, 0.0) - jnp.log(1.0 + jnp.exp(-jnp.abs(x)))
    reg = jnp.sum(regi_ref[...]) + jnp.sum(regu_ref[...])
    out_ref[0, 0] = _REG * reg - jnp.sum(ls)


_loss_call = pl.pallas_call(
    _loss_body,
    out_shape=jax.ShapeDtypeStruct((1, 1), jnp.float32),
    out_specs=pl.BlockSpec(memory_space=pltpu.SMEM),
)


def kernel(user, item_i, item_j, embed_user, embed_item):
    ei2 = _pair_rows(embed_item)
    svi, svj, regij = _sc_item(item_i, item_j, ei2)
    eu2 = _pair_rows(embed_user)
    pi, pj, regu = _sc_user(user, eu2, svi, svj)
    loss = _loss_call(pi.reshape(_B // 128, 128),
                      pj.reshape(_B // 128, 128),
                      regij.reshape(_NW * _L // 128, 128),
                      regu.reshape(_NW * _L // 128, 128))[0, 0]
    return (pi, pj, loss)


# speedup vs baseline: 1.0903x; 1.0686x over previous
import functools

import jax
import jax.numpy as jnp
from jax import lax
from jax.experimental import pallas as pl
from jax.experimental.pallas import tpu as pltpu
from jax.experimental.pallas import tpu_sc as plsc

_REG = 0.001
_B = 16384
_D = 64
_NC = 2
_NS = 16
_L = 16
_NW = _NC * _NS
_BPW = _B // _NW
_CHUNK = 128
_NCHUNK = _BPW // _CHUNK

_sc_mesh = plsc.VectorSubcoreMesh(
    core_axis_name="c", subcore_axis_name="s",
    num_cores=_NC, num_subcores=_NS)
_sc_params = pltpu.CompilerParams(
    needs_layout_passes=False, use_tc_tiling_on_sc=False)


def _shuffled_row(v):
    return lax.bitwise_or(
        lax.bitwise_or(
            lax.shift_left(lax.shift_right_logical(v, 4), 4),
            lax.shift_left(lax.bitwise_and(v, 7), 1)),
        lax.bitwise_and(lax.shift_right_logical(v, 3), 1))


def _sc_item_body(ii_hbm, ij_hbm, et_hbm,
                  svi_hbm, svj_hbm, reg_hbm,
                  iidx_v, jidx_v, ipr_v, jpr_v,
                  via, vja, vib, vjb, reg_v,
                  sem_a, sem_b, sem_w):
    c = lax.axis_index("c")
    s = lax.axis_index("s")
    wid = s * _NC + c
    base = wid * _BPW

    pltpu.sync_copy(ii_hbm.at[pl.ds(base, _BPW)], iidx_v)
    pltpu.sync_copy(ij_hbm.at[pl.ds(base, _BPW)], jidx_v)

    def prep(i, _):
        sl = pl.ds(i * _L, _L)
        ipr_v[sl] = _shuffled_row(iidx_v[sl])
        jpr_v[sl] = _shuffled_row(jidx_v[sl])
        return 0
    lax.fori_loop(0, _BPW // _L, prep, 0)

    slots = ((via, vja, sem_a), (vib, vjb, sem_b))

    def fire(k, slot):
        vi_v, vj_v, sem = slot
        sl = pl.ds(k * _CHUNK, _CHUNK)
        return (pltpu.async_copy(et_hbm.at[ipr_v.at[sl]], vi_v, sem),
                pltpu.async_copy(et_hbm.at[jpr_v.at[sl]], vj_v, sem))

    reg_acc = jnp.zeros((_L,), jnp.float32)
    inflight = fire(0, slots[0])
    writes = ()
    for k in range(_NCHUNK):
        vi_v, vj_v, _ = slots[k % 2]
        for cp in inflight:
            cp.wait()
        if k + 1 < _NCHUNK:
            inflight = fire(k + 1, slots[(k + 1) % 2])

        def sq(g, reg_acc, vi_v=vi_v, vj_v=vj_v):
            for l in range(_L):
                r = g * _L + l
                for cc in range(_D // _L):
                    col = pl.ds(cc * _L, _L)
                    vv = vi_v[r, col]
                    ww = vj_v[r, col]
                    reg_acc = reg_acc + vv * vv + ww * ww
            return reg_acc

        reg_acc = lax.fori_loop(0, _CHUNK // _L, sq, reg_acc)
        for cp in writes:
            cp.wait()
        dst = pl.ds(base + k * _CHUNK, _CHUNK)
        writes = (pltpu.async_copy(vi_v, svi_hbm.at[dst], sem_w),
                  pltpu.async_copy(vj_v, svj_hbm.at[dst], sem_w))
    for cp in writes:
        cp.wait()

    reg_v[...] = reg_acc
    pltpu.sync_copy(reg_v, reg_hbm.at[pl.ds(wid * _L, _L)])


_sc_item = functools.partial(
    pl.kernel,
    out_type=(
        jax.ShapeDtypeStruct((_B, _D), jnp.float32),
        jax.ShapeDtypeStruct((_B, _D), jnp.float32),
        jax.ShapeDtypeStruct((_NW * _L,), jnp.float32),
    ),
    mesh=_sc_mesh,
    compiler_params=_sc_params,
    scratch_types=[
        pltpu.VMEM((_BPW,), jnp.int32),
        pltpu.VMEM((_BPW,), jnp.int32),
        pltpu.VMEM((_BPW,), jnp.int32),
        pltpu.VMEM((_BPW,), jnp.int32),
        pltpu.VMEM((_CHUNK, _D), jnp.float32),
        pltpu.VMEM((_CHUNK, _D), jnp.float32),
        pltpu.VMEM((_CHUNK, _D), jnp.float32),
        pltpu.VMEM((_CHUNK, _D), jnp.float32),
        pltpu.VMEM((_L,), jnp.float32),
        pltpu.SemaphoreType.DMA,
        pltpu.SemaphoreType.DMA,
        pltpu.SemaphoreType.DMA,
    ],
)(_sc_item_body)


def _sc_user_body(user_hbm, eut_hbm, svi_hbm, svj_hbm,
                  pi_hbm, pj_hbm, reg_hbm,
                  uidx_v, upr_v,
                  ua, via, vja, ub, vib, vjb, pi_v, pj_v, reg_v,
                  sem_a, sem_b):
    c = lax.axis_index("c")
    s = lax.axis_index("s")
    wid = s * _NC + c
    base = wid * _BPW

    pltpu.sync_copy(user_hbm.at[pl.ds(base, _BPW)], uidx_v)

    def prep(i, _):
        sl = pl.ds(i * _L, _L)
        upr_v[sl] = _shuffled_row(uidx_v[sl])
        return 0
    lax.fori_loop(0, _BPW // _L, prep, 0)

    slots = ((ua, via, vja, sem_a), (ub, vib, vjb, sem_b))

    def fire(k, slot):
        u_v, vi_v, vj_v, sem = slot
        sl = pl.ds(k * _CHUNK, _CHUNK)
        ssl = pl.ds(base + k * _CHUNK, _CHUNK)
        return (pltpu.async_copy(eut_hbm.at[upr_v.at[sl]], u_v, sem),
                pltpu.async_copy(svi_hbm.at[ssl], vi_v, sem),
                pltpu.async_copy(svj_hbm.at[ssl], vj_v, sem))

    lane = lax.iota(jnp.int32, _L)
    reg_acc = jnp.zeros((_L,), jnp.float32)
    inflight = fire(0, slots[0])
    for k in range(_NCHUNK):
        u_v, vi_v, vj_v, _ = slots[k % 2]
        for cp in inflight:
            cp.wait()
        if k + 1 < _NCHUNK:
            inflight = fire(k + 1, slots[(k + 1) % 2])

        def group(g, reg_acc, u_v=u_v, vi_v=vi_v, vj_v=vj_v, k=k):
            acc_i = jnp.zeros((_L,), jnp.float32)
            acc_j = jnp.zeros((_L,), jnp.float32)
            for l in range(_L):
                r = g * _L + l
                ss_i = jnp.zeros((_L,), jnp.float32)
                ss_j = jnp.zeros((_L,), jnp.float32)
                for cc in range(_D // _L):
                    col = pl.ds(cc * _L, _L)
                    uu = u_v[r, col]
                    ss_i = ss_i + uu * vi_v[r, col]
                    ss_j = ss_j + uu * vj_v[r, col]
                    reg_acc = reg_acc + uu * uu
                acc_i = jnp.where(lane == l, jnp.sum(ss_i), acc_i)
                acc_j = jnp.where(lane == l, jnp.sum(ss_j), acc_j)
            row = pl.ds(k * _CHUNK + g * _L, _L)
            pi_v[row] = acc_i
            pj_v[row] = acc_j
            return reg_acc

        reg_acc = lax.fori_loop(0, _CHUNK // _L, group, reg_acc)

    reg_v[...] = reg_acc
    pltpu.sync_copy(pi_v, pi_hbm.at[pl.ds(base, _BPW)])
    pltpu.sync_copy(pj_v, pj_hbm.at[pl.ds(base, _BPW)])
    pltpu.sync_copy(reg_v, reg_hbm.at[pl.ds(wid * _L, _L)])


_sc_user = functools.partial(
    pl.kernel,
    out_type=(
        jax.ShapeDtypeStruct((_B,), jnp.float32),
        jax.ShapeDtypeStruct((_B,), jnp.float32),
        jax.ShapeDtypeStruct((_NW * _L,), jnp.float32),
    ),
    mesh=_sc_mesh,
    compiler_params=_sc_params,
    scratch_types=[
        pltpu.VMEM((_BPW,), jnp.int32),
        pltpu.VMEM((_BPW,), jnp.int32),
        pltpu.VMEM((_CHUNK, _D), jnp.float32),
        pltpu.VMEM((_CHUNK, _D), jnp.float32),
        pltpu.VMEM((_CHUNK, _D), jnp.float32),
        pltpu.VMEM((_CHUNK, _D), jnp.float32),
        pltpu.VMEM((_CHUNK, _D), jnp.float32),
        pltpu.VMEM((_CHUNK, _D), jnp.float32),
        pltpu.VMEM((_BPW,), jnp.float32),
        pltpu.VMEM((_BPW,), jnp.float32),
        pltpu.VMEM((_L,), jnp.float32),
        pltpu.SemaphoreType.DMA,
        pltpu.SemaphoreType.DMA,
    ],
)(_sc_user_body)


def _pair_body(t_ref, o_ref):
    x = jnp.swapaxes(t_ref[...], 0, 1)
    q = x.shape[0] // 16
    x4 = x.reshape(q, 2, 8, _D)
    out = jnp.concatenate([x4[:, 0], x4[:, 1]], axis=-1)
    o_ref[...] = out.reshape(q * 8 * 2 * _D)


def _linear_rows(table):
    n = table.shape[0]
    m = 8 * ((n + 15) // 16)
    blk = 8192
    grid = (m + blk - 1) // blk
    flat = pl.pallas_call(
        _pair_body,
        grid=(grid,),
        in_specs=[pl.BlockSpec((_D, 2 * blk), lambda i: (0, i))],
        out_specs=pl.BlockSpec((blk * 2 * _D,), lambda i: (i,)),
        out_shape=jax.ShapeDtypeStruct((m * 2 * _D,), table.dtype),
    )(table.T)
    return flat.reshape(2 * m, _D)


def _loss_body(pi_ref, pj_ref, regi_ref, regu_ref, out_ref):
    x = pi_ref[...] - pj_ref[...]
    ls = jnp.minimum(x, 0.0) - jnp.log(1.0 + jnp.exp(-jnp.abs(x)))
    reg = jnp.sum(regi_ref[...]) + jnp.sum(regu_ref[...])
    out_ref[0, 0] = _REG * reg - jnp.sum(ls)


_loss_call = pl.pallas_call(
    _loss_body,
    out_shape=jax.ShapeDtypeStruct((1, 1), jnp.float32),
    out_specs=pl.BlockSpec(memory_space=pltpu.SMEM),
)


def kernel(user, item_i, item_j, embed_user, embed_item):
    ei2 = _linear_rows(embed_item)
    svi, svj, regij = _sc_item(item_i, item_j, ei2)
    eu2 = _linear_rows(embed_user)
    pi, pj, regu = _sc_user(user, eu2, svi, svj)
    loss = _loss_call(pi.reshape(_B // 128, 128),
                      pj.reshape(_B // 128, 128),
                      regij.reshape(_NW * _L // 128, 128),
                      regu.reshape(_NW * _L // 128, 128))[0, 0]
    return (pi, pj, loss)
